# baseline (speedup 1.0000x reference)
import functools
import jax, jax.numpy as jnp
from jax import lax
from jax.experimental import pallas as pl
from jax.experimental.pallas import tpu as pltpu
from jax.experimental.pallas import tpu_sc as plsc


def kernel(tokens, W_E):
    mesh = plsc.VectorSubcoreMesh(core_axis_name="c", subcore_axis_name="s")
    w3 = W_E.reshape(128, 8, 100000)

    @functools.partial(
        pl.kernel,
        mesh=mesh,
        compiler_params=pltpu.CompilerParams(use_tc_tiling_on_sc=False),
        out_type=jax.ShapeDtypeStruct((16384, 128, 8), jnp.float32),
        scratch_types=[
            pltpu.VMEM((128, 8), jnp.float32),
            pltpu.SemaphoreType.DMA,
        ],
    )
    def k(w_hbm, out_hbm, buf, sem):
        wid = lax.axis_index("s") * 2 + lax.axis_index("c")
        pltpu.async_copy(w_hbm.at[:, :, 0], buf, sem).wait()
        pltpu.sync_copy(buf, out_hbm.at[wid])

    flat = k(w3)
    return flat.reshape(4, 4096, 1024)


# speedup vs baseline: 13.9205x; 13.9205x over previous
import functools

import jax
import jax.numpy as jnp
from jax import lax
from jax.experimental import pallas as pl
from jax.experimental.pallas import tpu as pltpu
from jax.experimental.pallas import tpu_sc as plsc

D_MODEL = 1024
D_VOCAB = 100000
NUM_TOKENS = 4 * 4096

_NC = 2
_NS = 16
_NW = _NC * _NS
_B_PER_W = NUM_TOKENS // _NW
_CG = 32
_NG = _B_PER_W // _CG


def _sc_gather(table, tokens_flat):
    mesh = plsc.VectorSubcoreMesh(core_axis_name="c", subcore_axis_name="s")

    @functools.partial(
        pl.kernel,
        mesh=mesh,
        out_type=jax.ShapeDtypeStruct((NUM_TOKENS, D_MODEL), jnp.float32),
        scratch_types=[
            pltpu.VMEM((_B_PER_W,), jnp.int32),
            pltpu.VMEM((_CG, D_MODEL), jnp.float32),
            pltpu.VMEM((_CG, D_MODEL), jnp.float32),
            pltpu.SemaphoreType.DMA,
            pltpu.SemaphoreType.DMA,
            pltpu.SemaphoreType.DMA,
            pltpu.SemaphoreType.DMA,
        ],
    )
    def k(table_hbm, idx_hbm, out_hbm, idx_v, buf_a, buf_b, sem_ga, sem_gb,
          sem_wa, sem_wb):
        wid = lax.axis_index("s") * _NC + lax.axis_index("c")
        base = wid * _B_PER_W
        pltpu.sync_copy(idx_hbm.at[pl.ds(base, _B_PER_W)], idx_v)

        bufs = (buf_a, buf_b)
        gsems = (sem_ga, sem_gb)
        wsems = (sem_wa, sem_wb)

        def gather(g, buf, sem):
            return pltpu.async_copy(
                table_hbm.at[idx_v.at[pl.ds(g * _CG, _CG)]], buf, sem)

        gathers = [None, None]
        writes = [None, None]
        gathers[0] = gather(0, bufs[0], gsems[0])
        for g in range(_NG):
            s = g % 2
            if g + 1 < _NG:
                if writes[1 - s] is not None:
                    writes[1 - s].wait()
                gathers[1 - s] = gather(g + 1, bufs[1 - s], gsems[1 - s])
            gathers[s].wait()
            writes[s] = pltpu.async_copy(
                bufs[s], out_hbm.at[pl.ds(base + g * _CG, _CG)], wsems[s])
        writes[0].wait()
        writes[1].wait()

    return k(table, tokens_flat)


def kernel(tokens, W_E):
    w_t = jnp.transpose(W_E)
    flat = _sc_gather(w_t, tokens.reshape(NUM_TOKENS))
    return flat.reshape(tokens.shape[0], tokens.shape[1], D_MODEL)


# speedup vs baseline: 14.0478x; 1.0091x over previous
import functools

import jax
import jax.numpy as jnp
from jax import lax
from jax.experimental import pallas as pl
from jax.experimental.pallas import tpu as pltpu
from jax.experimental.pallas import tpu_sc as plsc

D_MODEL = 1024
D_VOCAB = 100000
NUM_TOKENS = 4 * 4096

_NC = 2
_NS = 16
_NW = _NC * _NS
_B_PER_W = NUM_TOKENS // _NW
_CG = 32
_NG = _B_PER_W // _CG


def _sc_gather(table, tokens_flat):
    mesh = plsc.VectorSubcoreMesh(core_axis_name="c", subcore_axis_name="s")

    @functools.partial(
        pl.kernel,
        mesh=mesh,
        out_type=jax.ShapeDtypeStruct((NUM_TOKENS, D_MODEL), jnp.float32),
        scratch_types=[
            pltpu.VMEM((_B_PER_W,), jnp.int32),
            pltpu.VMEM((_CG, D_MODEL), jnp.float32),
            pltpu.VMEM((_CG, D_MODEL), jnp.float32),
            pltpu.VMEM((_CG, D_MODEL), jnp.float32),
            pltpu.SemaphoreType.DMA,
            pltpu.SemaphoreType.DMA,
            pltpu.SemaphoreType.DMA,
            pltpu.SemaphoreType.DMA,
            pltpu.SemaphoreType.DMA,
            pltpu.SemaphoreType.DMA,
        ],
    )
    def k(table_hbm, idx_hbm, out_hbm, idx_v, buf_a, buf_b, buf_c, sem_ga,
          sem_gb, sem_gc, sem_wa, sem_wb, sem_wc):
        wid = lax.axis_index("s") * _NC + lax.axis_index("c")
        base = wid * _B_PER_W
        pltpu.sync_copy(idx_hbm.at[pl.ds(base, _B_PER_W)], idx_v)

        bufs = (buf_a, buf_b, buf_c)
        gsems = (sem_ga, sem_gb, sem_gc)
        wsems = (sem_wa, sem_wb, sem_wc)

        def gather(g, buf, sem):
            return pltpu.async_copy(
                table_hbm.at[idx_v.at[pl.ds(g * _CG, _CG)]], buf, sem)

        nb = 3
        gathers = [None] * nb
        writes = [None] * nb
        gathers[0] = gather(0, bufs[0], gsems[0])
        gathers[1] = gather(1, bufs[1], gsems[1])
        for g in range(_NG):
            s = g % nb
            n = (g + 2) % nb
            if g + 2 < _NG:
                if writes[n] is not None:
                    writes[n].wait()
                gathers[n] = gather(g + 2, bufs[n], gsems[n])
            gathers[s].wait()
            writes[s] = pltpu.async_copy(
                bufs[s], out_hbm.at[pl.ds(base + g * _CG, _CG)], wsems[s])
        for wh in writes:
            wh.wait()

    return k(table, tokens_flat)


def kernel(tokens, W_E):
    w_t = jnp.transpose(W_E)
    flat = _sc_gather(w_t, tokens.reshape(NUM_TOKENS))
    return flat.reshape(tokens.shape[0], tokens.shape[1], D_MODEL)


# speedup vs baseline: 14.0641x; 1.0012x over previous
import functools

import jax
import jax.numpy as jnp
from jax import lax
from jax.experimental import pallas as pl
from jax.experimental.pallas import tpu as pltpu
from jax.experimental.pallas import tpu_sc as plsc

D_MODEL = 1024
D_VOCAB = 100000
NUM_TOKENS = 4 * 4096

_NC = 2
_NS = 16
_NW = _NC * _NS
_B_PER_W = NUM_TOKENS // _NW
_CG = 32
_NG = _B_PER_W // _CG


def _sc_gather(table, tokens_flat):
    mesh = plsc.VectorSubcoreMesh(core_axis_name="c", subcore_axis_name="s")

    @functools.partial(
        pl.kernel,
        mesh=mesh,
        compiler_params=pltpu.CompilerParams(skip_device_barrier=True),
        out_type=jax.ShapeDtypeStruct((NUM_TOKENS, D_MODEL), jnp.float32),
        scratch_types=[
            pltpu.VMEM((_B_PER_W,), jnp.int32),
            pltpu.VMEM((_CG, D_MODEL), jnp.float32),
            pltpu.VMEM((_CG, D_MODEL), jnp.float32),
            pltpu.VMEM((_CG, D_MODEL), jnp.float32),
            pltpu.SemaphoreType.DMA,
            pltpu.SemaphoreType.DMA,
            pltpu.SemaphoreType.DMA,
            pltpu.SemaphoreType.DMA,
            pltpu.SemaphoreType.DMA,
            pltpu.SemaphoreType.DMA,
        ],
    )
    def k(table_hbm, idx_hbm, out_hbm, idx_v, buf_a, buf_b, buf_c, sem_ga,
          sem_gb, sem_gc, sem_wa, sem_wb, sem_wc):
        wid = lax.axis_index("s") * _NC + lax.axis_index("c")
        base = wid * _B_PER_W
        pltpu.sync_copy(idx_hbm.at[pl.ds(base, _B_PER_W)], idx_v)

        bufs = (buf_a, buf_b, buf_c)
        gsems = (sem_ga, sem_gb, sem_gc)
        wsems = (sem_wa, sem_wb, sem_wc)

        def gather(g, buf, sem):
            return pltpu.async_copy(
                table_hbm.at[idx_v.at[pl.ds(g * _CG, _CG)]], buf, sem)

        nb = 3
        gathers = [None] * nb
        writes = [None] * nb
        gathers[0] = gather(0, bufs[0], gsems[0])
        gathers[1] = gather(1, bufs[1], gsems[1])
        for g in range(_NG):
            s = g % nb
            n = (g + 2) % nb
            if g + 2 < _NG:
                if writes[n] is not None:
                    writes[n].wait()
                gathers[n] = gather(g + 2, bufs[n], gsems[n])
            gathers[s].wait()
            writes[s] = pltpu.async_copy(
                bufs[s], out_hbm.at[pl.ds(base + g * _CG, _CG)], wsems[s])
        for wh in writes:
            wh.wait()

    return k(table, tokens_flat)


def kernel(tokens, W_E):
    w_t = jnp.transpose(W_E)
    flat = _sc_gather(w_t, tokens.reshape(NUM_TOKENS))
    return flat.reshape(tokens.shape[0], tokens.shape[1], D_MODEL)


# speedup vs baseline: 14.1456x; 1.0058x over previous
import functools

import jax
import jax.numpy as jnp
from jax import lax
from jax.experimental import pallas as pl
from jax.experimental.pallas import tpu as pltpu
from jax.experimental.pallas import tpu_sc as plsc

D_MODEL = 1024
D_VOCAB = 100000
NUM_TOKENS = 4 * 4096

_NC = 2
_NS = 16
_NW = _NC * _NS
_B_PER_W = NUM_TOKENS // _NW
_CG = 32
_NG = _B_PER_W // _CG


def _sc_gather(table, tokens_flat):
    mesh = plsc.VectorSubcoreMesh(core_axis_name="c", subcore_axis_name="s")

    @functools.partial(
        pl.kernel,
        mesh=mesh,
        out_type=jax.ShapeDtypeStruct((NUM_TOKENS, D_MODEL), jnp.float32),
        scratch_types=[
            pltpu.VMEM((_B_PER_W,), jnp.int32),
            pltpu.VMEM((_CG, D_MODEL), jnp.float32),
            pltpu.VMEM((_CG, D_MODEL), jnp.float32),
            pltpu.VMEM((_CG, D_MODEL), jnp.float32),
            pltpu.SemaphoreType.DMA,
            pltpu.SemaphoreType.DMA,
            pltpu.SemaphoreType.DMA,
            pltpu.SemaphoreType.DMA,
            pltpu.SemaphoreType.DMA,
            pltpu.SemaphoreType.DMA,
        ],
    )
    def k(table_hbm, idx_hbm, out_hbm, idx_v, buf_a, buf_b, buf_c, sem_ga,
          sem_gb, sem_gc, sem_wa, sem_wb, sem_wc):
        wid = lax.axis_index("s") * _NC + lax.axis_index("c")
        base = wid * _B_PER_W
        pltpu.sync_copy(idx_hbm.at[pl.ds(base, _B_PER_W)], idx_v)

        bufs = (buf_a, buf_b, buf_c)
        gsems = (sem_ga, sem_gb, sem_gc)
        wsems = (sem_wa, sem_wb, sem_wc)

        def gather(g, buf, sem):
            return pltpu.async_copy(
                table_hbm.at[idx_v.at[pl.ds(g * _CG, _CG)]], buf, sem)

        nb = 3
        gathers = [None] * nb
        writes = [None] * nb
        gathers[0] = gather(0, bufs[0], gsems[0])
        gathers[1] = gather(1, bufs[1], gsems[1])
        for g in range(_NG):
            s = g % nb
            n = (g + 2) % nb
            if g + 2 < _NG:
                if writes[n] is not None:
                    writes[n].wait()
                gathers[n] = gather(g + 2, bufs[n], gsems[n])
            gathers[s].wait()
            writes[s] = pltpu.async_copy(
                bufs[s], out_hbm.at[pl.ds(base + g * _CG, _CG)], wsems[s])
        for wh in writes:
            wh.wait()

    return k(table, tokens_flat)


def kernel(tokens, W_E):
    w_t = jnp.transpose(W_E)
    flat = _sc_gather(w_t, tokens.reshape(NUM_TOKENS))
    return flat.reshape(tokens.shape[0], tokens.shape[1], D_MODEL)
